# baseline (speedup 1.0000x reference)
import functools

import jax
import jax.numpy as jnp
from jax import lax
from jax.experimental import pallas as pl
from jax.experimental.pallas import tpu as pltpu
from jax.experimental.pallas import tpu_sc as plsc

_B, _C, _H, _W = 4, 256, 32, 32
_D = _C // 2
_HW = _H * _W
_NEG = -1e9
_EPS = 1e-8
_THR = 0.75


def _match_body(x_ref, mcol_ref, out01_ref, idx_ref, table_ref):
    fw = x_ref[0, :_D]
    lw = x_ref[0, _D:]
    fl_col = (mcol_ref[0] > _THR).astype(jnp.float32)

    kn = fw / (jnp.sqrt(jnp.sum(fw * fw, axis=0, keepdims=True)) + _EPS)
    qn = lw / (jnp.sqrt(jnp.sum(lw * lw, axis=0, keepdims=True)) + _EPS)

    cosT = lax.dot_general(kn, qn, (((0,), (0,)), ((), ())),
                           preferred_element_type=jnp.float32)
    cosT = cosT + fl_col * _NEG

    m = jnp.max(cosT, axis=0, keepdims=True)
    iot = lax.broadcasted_iota(jnp.int32, (_HW, _HW), 0)
    idx = jnp.min(jnp.where(cosT == m, iot, _HW), axis=0, keepdims=True)

    b = pl.program_id(0)
    idx_ref[0] = idx + b * _HW
    out01_ref[0, :_D] = fw
    out01_ref[0, _D:] = lw
    table_ref[0] = jnp.swapaxes(fw, 0, 1)


def _sc_gather(table_ref, idx_ref, out_ref, idx_v, rows_v, sem):
    nc = 2
    wid = lax.axis_index("s") * nc + lax.axis_index("c")
    rows_per_w = (_B * _HW) // 32
    base = wid * rows_per_w
    pltpu.sync_copy(idx_ref.at[pl.ds(base, rows_per_w)], idx_v)
    pltpu.async_copy(table_ref.at[idx_v], rows_v, sem).wait()
    pltpu.sync_copy(rows_v, out_ref.at[pl.ds(base, rows_per_w)])


@functools.partial(jax.jit, static_argnames=())
def kernel(input, mask):
    b, c, h, w = input.shape
    d = c // 2
    hw = h * w
    x3 = input.reshape(b, c, hw)
    mask_col = mask.reshape(b, hw, 1)

    out01, idx3, table = pl.pallas_call(
        _match_body,
        grid=(b,),
        in_specs=[
            pl.BlockSpec((1, c, hw), lambda i: (i, 0, 0)),
            pl.BlockSpec((1, hw, 1), lambda i: (i, 0, 0)),
        ],
        out_specs=[
            pl.BlockSpec((1, 2 * d, hw), lambda i: (i, 0, 0)),
            pl.BlockSpec((1, 1, hw), lambda i: (i, 0, 0)),
            pl.BlockSpec((1, hw, d), lambda i: (i, 0, 0)),
        ],
        out_shape=[
            jax.ShapeDtypeStruct((b, 2 * d, hw), jnp.float32),
            jax.ShapeDtypeStruct((b, 1, hw), jnp.int32),
            jax.ShapeDtypeStruct((b, hw, d), jnp.float32),
        ],
        compiler_params=pltpu.CompilerParams(
            dimension_semantics=("parallel",)),
    )(x3, mask_col)

    mesh = plsc.VectorSubcoreMesh(core_axis_name="c", subcore_axis_name="s")
    rows_per_w = (b * hw) // 32
    gather = functools.partial(
        pl.kernel,
        mesh=mesh,
        out_type=jax.ShapeDtypeStruct((b * hw, d), jnp.float32),
        scratch_types=[
            pltpu.VMEM((rows_per_w,), jnp.int32),
            pltpu.VMEM((rows_per_w, d), jnp.float32),
            pltpu.SemaphoreType.DMA,
        ],
    )(_sc_gather)
    shift_rows = gather(table.reshape(b * hw, d), idx3.reshape(b * hw))

    flag = (mask.reshape(b, hw) > _THR).astype(jnp.float32)
    shift = (shift_rows.reshape(b, hw, d) * flag[:, :, None]).transpose(0, 2, 1)
    final_out = jnp.concatenate([out01, shift], axis=1).reshape(b, 3 * d, h, w)
    inner_feat = input[:, d:]
    return final_out, inner_feat


# speedup vs baseline: 1.7086x; 1.7086x over previous
import functools

import jax
import jax.numpy as jnp
from jax import lax
from jax.experimental import pallas as pl
from jax.experimental.pallas import tpu as pltpu

_B, _C, _H, _W = 4, 256, 32, 32
_D = _C // 2
_HW = _H * _W
_NEG = -1e9
_EPS = 1e-8
_THR = 0.75


def _shift_body(x_ref, mrow_ref, mcol_ref, out_ref):
    fw = x_ref[0, :_D]
    lw = x_ref[0, _D:]
    fl_row = (mrow_ref[0] > _THR).astype(jnp.float32)
    fl_col = mcol_ref[0] > _THR

    kn = fw / (jnp.sqrt(jnp.sum(fw * fw, axis=0, keepdims=True)) + _EPS)
    qn = lw / (jnp.sqrt(jnp.sum(lw * lw, axis=0, keepdims=True)) + _EPS)

    cos = lax.dot_general(qn, kn, (((0,), (0,)), ((), ())),
                          preferred_element_type=jnp.float32)
    cos = cos + fl_row * _NEG

    m = jnp.max(cos, axis=1, keepdims=True)
    iot = lax.broadcasted_iota(jnp.int32, (_HW, _HW), 1)
    idx = jnp.min(jnp.where(cos == m, iot, _HW), axis=1, keepdims=True)

    onehot = jnp.where((iot == idx) & fl_col, 1.0, 0.0)
    shift = lax.dot_general(fw, onehot, (((1,), (1,)), ((), ())),
                            preferred_element_type=jnp.float32)
    out_ref[0, :_D] = fw
    out_ref[0, _D:2 * _D] = lw
    out_ref[0, 2 * _D:] = shift


@functools.partial(jax.jit, static_argnames=())
def kernel(input, mask):
    b, c, h, w = input.shape
    d = c // 2
    hw = h * w
    x3 = input.reshape(b, c, hw)
    mask_row = mask.reshape(b, 1, hw)
    mask_col = mask.reshape(b, hw, 1)

    final_d = pl.pallas_call(
        _shift_body,
        grid=(b,),
        in_specs=[
            pl.BlockSpec((1, c, hw), lambda i: (i, 0, 0)),
            pl.BlockSpec((1, 1, hw), lambda i: (i, 0, 0)),
            pl.BlockSpec((1, hw, 1), lambda i: (i, 0, 0)),
        ],
        out_specs=pl.BlockSpec((1, 3 * d, hw), lambda i: (i, 0, 0)),
        out_shape=jax.ShapeDtypeStruct((b, 3 * d, hw), jnp.float32),
        compiler_params=pltpu.CompilerParams(
            dimension_semantics=("parallel",)),
    )(x3, mask_row, mask_col)

    final_out = final_d.reshape(b, 3 * d, h, w)
    inner_feat = input[:, d:]
    return final_out, inner_feat
